# baseline (speedup 1.0000x reference)
import functools

import jax
import jax.numpy as jnp
from jax import lax
from jax.experimental import pallas as pl
from jax.experimental.pallas import tpu as pltpu
from jax.experimental.pallas import tpu_sc as plsc

N = 5000
L = 24
D = 256
MAX_DIST = 0.3
MAX_GAP = 10
IOU_T = 0.5

TR = 200
NP = 5120
CC = 512

NW = 32
RPW = 160
KB = 16


def _tc_body(scal_ref, boxes_ref, bT_ref, fidx_ref, mask_ref, b23_ref, qp_ref,
             upd_out, s23_out, fidx_out, mask_out, accum_out):
    bx = boxes_ref[...]
    cxr, cyr, wr, hr = bx[:, 0:1], bx[:, 1:2], bx[:, 2:3], bx[:, 3:4]
    x1r = cxr - 0.5 * wr
    y1r = cyr - 0.5 * hr
    x2r = cxr + 0.5 * wr
    y2r = cyr + 0.5 * hr
    area_r = (x2r - x1r) * (y2r - y1r)

    acc = jnp.full((TR, CC), -jnp.inf, dtype=jnp.float32)
    for c in range(NP // CC):
        sl = pl.ds(c * CC, CC)
        cxc = bT_ref[0:1, sl]
        cyc = bT_ref[1:2, sl]
        wc = bT_ref[2:3, sl]
        hc = bT_ref[3:4, sl]
        x1c = cxc - 0.5 * wc
        y1c = cyc - 0.5 * hc
        x2c = cxc + 0.5 * wc
        y2c = cyc + 0.5 * hc
        area_c = (x2c - x1c) * (y2c - y1c)
        ltx = jnp.maximum(x1r, x1c)
        lty = jnp.maximum(y1r, y1c)
        rbx = jnp.minimum(x2r, x2c)
        rby = jnp.minimum(y2r, y2c)
        inter = jnp.maximum(rbx - ltx, 0.0) * (rby - lty)
        score = 3.0 * inter - (area_r + area_c)
        occm = y2c < y2r
        score = jnp.where(occm, score, -jnp.inf)
        acc = jnp.maximum(acc, score)
    occluded = jnp.max(acc, axis=1, keepdims=True) > 0.0

    sc = scal_ref[...]
    f = sc[:, 0:1]
    dist = sc[:, 1:2]
    score_q = sc[:, 2:3]
    fr = fidx_ref[...]
    last = fr[:, L - 1:L].astype(jnp.float32)
    upd = ((f - last > MAX_GAP) | (dist > MAX_DIST)) & (~occluded) & (score_q > 0.8)

    upd_out[...] = upd.astype(jnp.int32)
    accum_out[...] = jnp.where(upd, 0.0, dist)

    new_f = jnp.concatenate([fr[:, 1:], f.astype(jnp.int32)], axis=1)
    fidx_out[...] = jnp.where(upd, new_f, fr)

    mr = mask_ref[...].astype(jnp.int32)
    new_m = jnp.concatenate([mr[:, 1:], jnp.zeros((TR, 1), jnp.int32)], axis=1)
    mask_out[...] = jnp.where(upd, new_m, mr).astype(jnp.uint8)

    b23 = b23_ref[...]
    qp = qp_ref[...]
    s23_out[...] = jnp.where(upd, qp, b23)


def _tc_stage(scal, pred_boxes, bT, mem_frames_idx, mask_u8, bank23, query_pos):
    f32 = jnp.float32
    grid = (N // TR,)
    return pl.pallas_call(
        _tc_body,
        grid=grid,
        in_specs=[
            pl.BlockSpec((TR, 4), lambda i: (i, 0)),
            pl.BlockSpec((TR, 4), lambda i: (i, 0)),
            pl.BlockSpec((8, NP), lambda i: (0, 0)),
            pl.BlockSpec((TR, L), lambda i: (i, 0)),
            pl.BlockSpec((TR, L), lambda i: (i, 0)),
            pl.BlockSpec((TR, D), lambda i: (i, 0)),
            pl.BlockSpec((TR, D), lambda i: (i, 0)),
        ],
        out_specs=[
            pl.BlockSpec((TR, 1), lambda i: (i, 0)),
            pl.BlockSpec((TR, D), lambda i: (i, 0)),
            pl.BlockSpec((TR, L), lambda i: (i, 0)),
            pl.BlockSpec((TR, L), lambda i: (i, 0)),
            pl.BlockSpec((TR, 1), lambda i: (i, 0)),
        ],
        out_shape=[
            jax.ShapeDtypeStruct((N, 1), jnp.int32),
            jax.ShapeDtypeStruct((N, D), f32),
            jax.ShapeDtypeStruct((N, L), jnp.int32),
            jax.ShapeDtypeStruct((N, L), jnp.uint8),
            jax.ShapeDtypeStruct((N, 1), f32),
        ],
        compiler_params=pltpu.CompilerParams(
            dimension_semantics=("parallel",)),
    )(scal, pred_boxes, bT, mem_frames_idx, mask_u8, bank23, query_pos)


@functools.partial(
    pl.kernel,
    out_type=jax.ShapeDtypeStruct((N * L, D), jnp.float32),
    mesh=plsc.VectorSubcoreMesh(core_axis_name="c", subcore_axis_name="s"),
    compiler_params=pltpu.CompilerParams(use_tc_tiling_on_sc=False),
    scratch_types=[
        pltpu.VMEM((RPW,), jnp.int32),
        pltpu.SemaphoreType.DMA,
        pltpu.SemaphoreType.DMA,
    ],
)
def _sc_apply(upd_hbm, bank_hbm, s23_hbm, out_hbm, upd_v, sem_a, sem_b):
    wid = lax.axis_index("s") * 2 + lax.axis_index("c")
    base = jnp.minimum(wid * RPW, N - RPW)
    pltpu.sync_copy(upd_hbm.at[pl.ds(base, RPW)], upd_v)

    def batch(b, _):
        vec = upd_v[pl.ds(b * KB, KB)]
        handles = []
        for j in range(KB):
            i = b * KB + j
            s = vec[j]
            r0 = (base + i) * L
            h1 = pltpu.async_copy(bank_hbm.at[pl.ds(r0 + s, L - 1)],
                                  out_hbm.at[pl.ds(r0, L - 1)], sem_a)
            h2 = pltpu.async_copy(s23_hbm.at[pl.ds(base + i, 1)],
                                  out_hbm.at[pl.ds(r0 + L - 1, 1)], sem_b)
            handles.append((h1, h2))
        for h1, h2 in handles:
            h1.wait()
            h2.wait()
        return 0

    lax.fori_loop(0, RPW // KB, batch, 0)


def kernel(frame_idx, mem_frames_idx, accum_dist, pred_boxes, scores, mem_bank,
           mem_padding_mask, query_pos):
    f32 = jnp.float32
    scal = jnp.stack(
        [frame_idx.astype(f32), accum_dist, scores,
         jnp.zeros_like(accum_dist)], axis=1)
    bT = jnp.zeros((8, NP), f32).at[0:4, 0:N].set(pred_boxes.T)
    mask_u8 = mem_padding_mask.view(jnp.uint8)

    upd, s23, fidx_out, mask_out, accum_out = _tc_stage(
        scal, pred_boxes, bT, mem_frames_idx, mask_u8, mem_bank[:, L - 1, :],
        query_pos)

    bank_flat = mem_bank.reshape(N * L, D)
    bank_out = _sc_apply(upd.reshape(N), bank_flat, s23)

    return (bank_out.reshape(N, L, D), fidx_out, mask_out.view(jnp.bool_),
            accum_out.reshape(N))


# speedup vs baseline: 35.7771x; 35.7771x over previous
import jax
import jax.numpy as jnp
from jax.experimental import pallas as pl
from jax.experimental.pallas import tpu as pltpu

N = 5000
L = 24
D = 256
MAX_DIST = 0.3
MAX_GAP = 10
IOU_T = 0.5

TR = 200
NP = 5120
CC = 512


def _body(scal_ref, boxes_ref, bT_ref, fidx_ref, bank_ref, mask_ref, qp_ref,
          bank_out, fidx_out, mask_out, accum_out):
    bx = boxes_ref[...]
    cxr, cyr, wr, hr = bx[:, 0:1], bx[:, 1:2], bx[:, 2:3], bx[:, 3:4]
    x1r = cxr - 0.5 * wr
    y1r = cyr - 0.5 * hr
    x2r = cxr + 0.5 * wr
    y2r = cyr + 0.5 * hr
    area_r = (x2r - x1r) * (y2r - y1r)

    acc = jnp.full((TR, CC), -jnp.inf, dtype=jnp.float32)
    for c in range(NP // CC):
        sl = pl.ds(c * CC, CC)
        cxc = bT_ref[0:1, sl]
        cyc = bT_ref[1:2, sl]
        wc = bT_ref[2:3, sl]
        hc = bT_ref[3:4, sl]
        x1c = cxc - 0.5 * wc
        y1c = cyc - 0.5 * hc
        x2c = cxc + 0.5 * wc
        y2c = cyc + 0.5 * hc
        area_c = (x2c - x1c) * (y2c - y1c)
        ltx = jnp.maximum(x1r, x1c)
        lty = jnp.maximum(y1r, y1c)
        rbx = jnp.minimum(x2r, x2c)
        rby = jnp.minimum(y2r, y2c)
        inter = jnp.maximum(rbx - ltx, 0.0) * (rby - lty)
        score = 3.0 * inter - (area_r + area_c)
        occm = y2c < y2r
        score = jnp.where(occm, score, -jnp.inf)
        acc = jnp.maximum(acc, score)
    occluded = jnp.max(acc, axis=1, keepdims=True) > 0.0

    sc = scal_ref[...]
    f = sc[:, 0:1]
    dist = sc[:, 1:2]
    score_q = sc[:, 2:3]
    fr = fidx_ref[...]
    last = fr[:, L - 1:L].astype(jnp.float32)
    upd = ((f - last > MAX_GAP) | (dist > MAX_DIST)) & (~occluded) & (score_q > 0.8)

    accum_out[...] = jnp.where(upd, 0.0, dist)

    new_f = jnp.concatenate([fr[:, 1:], f.astype(jnp.int32)], axis=1)
    fidx_out[...] = jnp.where(upd, new_f, fr)

    mr = mask_ref[...].astype(jnp.int32)
    new_m = jnp.concatenate([mr[:, 1:], jnp.zeros((TR, 1), jnp.int32)], axis=1)
    mask_out[...] = jnp.where(upd, new_m, mr).astype(jnp.uint8)

    bk = bank_ref[...]
    qp = qp_ref[...]
    new_b = jnp.concatenate([bk[:, 1:, :], qp[:, None, :]], axis=1)
    bank_out[...] = jnp.where(upd[:, :, None], new_b, bk)


def kernel(frame_idx, mem_frames_idx, accum_dist, pred_boxes, scores, mem_bank,
           mem_padding_mask, query_pos):
    f32 = jnp.float32
    scal = jnp.stack(
        [frame_idx.astype(f32), accum_dist, scores,
         jnp.zeros_like(accum_dist)], axis=1)
    bT = jnp.zeros((8, NP), f32).at[0:4, 0:N].set(pred_boxes.T)
    mask_i = mem_padding_mask.view(jnp.uint8)

    grid = (N // TR,)
    out = pl.pallas_call(
        _body,
        grid=grid,
        in_specs=[
            pl.BlockSpec((TR, 4), lambda i: (i, 0)),
            pl.BlockSpec((TR, 4), lambda i: (i, 0)),
            pl.BlockSpec((8, NP), lambda i: (0, 0)),
            pl.BlockSpec((TR, L), lambda i: (i, 0)),
            pl.BlockSpec((TR, L, D), lambda i: (i, 0, 0)),
            pl.BlockSpec((TR, L), lambda i: (i, 0)),
            pl.BlockSpec((TR, D), lambda i: (i, 0)),
        ],
        out_specs=[
            pl.BlockSpec((TR, L, D), lambda i: (i, 0, 0)),
            pl.BlockSpec((TR, L), lambda i: (i, 0)),
            pl.BlockSpec((TR, L), lambda i: (i, 0)),
            pl.BlockSpec((TR, 1), lambda i: (i, 0)),
        ],
        out_shape=[
            jax.ShapeDtypeStruct((N, L, D), f32),
            jax.ShapeDtypeStruct((N, L), jnp.int32),
            jax.ShapeDtypeStruct((N, L), jnp.uint8),
            jax.ShapeDtypeStruct((N, 1), f32),
        ],
        compiler_params=pltpu.CompilerParams(
            dimension_semantics=("parallel",),
            allow_input_fusion=[True] * 7),
    )(scal, pred_boxes, bT, mem_frames_idx, mem_bank, mask_i, query_pos)

    bank_out, fidx_out, mask_out, accum_out = out
    return (bank_out, fidx_out, mask_out.view(jnp.bool_),
            accum_out.reshape(N))
